# initial kernel scaffold (unmeasured)
import jax
import jax.numpy as jnp
from jax import lax
from jax.experimental import pallas as pl
from jax.experimental.pallas import tpu as pltpu

T_LOC = 512
D = 1024
F = 2048
E_LOC = 4
E = 8


def _top2_weights(gates):
    col = lax.broadcasted_iota(jnp.int32, gates.shape, 1)
    a1 = jnp.argmax(gates, axis=1)[:, None]
    m1 = col == a1
    v1 = jnp.max(gates, axis=1, keepdims=True)
    g2 = jnp.where(m1, -1e30, gates)
    a2 = jnp.argmax(g2, axis=1)[:, None]
    m2 = col == a2
    v2 = jnp.max(g2, axis=1, keepdims=True)
    z = jnp.exp(v2 - v1)
    w_top = 1.0 / (1.0 + z)
    w_sec = z / (1.0 + z)
    zero = jnp.zeros_like(gates)
    return jnp.where(m1, w_top, zero) + jnp.where(m2, w_sec, zero)


def kernel(x, router, W1, W2):
    def body(
        x_ref, r_ref, w1_ref, w2_ref, out_ref,
        xrem_ref, rrem_ref, wdl_ref, wdr_ref,
        accl_ref, accr_ref, pp_ref, send_sems, recv_sems,
    ):
        e = pl.program_id(0)
        my_x = lax.axis_index("x")
        my_y = lax.axis_index("y")
        peer = (my_x, 1 - my_y)

        @pl.when(e == 0)
        def _dispatch():
            barrier = pltpu.get_barrier_semaphore()
            pl.semaphore_signal(
                barrier, inc=1, device_id=peer,
                device_id_type=pl.DeviceIdType.MESH,
            )
            pl.semaphore_wait(barrier, 1)

            x_rdma = pltpu.make_async_remote_copy(
                src_ref=x_ref, dst_ref=xrem_ref,
                send_sem=send_sems.at[0], recv_sem=recv_sems.at[0],
                device_id=peer, device_id_type=pl.DeviceIdType.MESH,
            )
            r_rdma = pltpu.make_async_remote_copy(
                src_ref=r_ref, dst_ref=rrem_ref,
                send_sem=send_sems.at[1], recv_sem=recv_sems.at[1],
                device_id=peer, device_id_type=pl.DeviceIdType.MESH,
            )
            x_rdma.start()
            r_rdma.start()
            x_rdma.wait()
            r_rdma.wait()

            rfull = jnp.concatenate([r_ref[...], rrem_ref[...]], axis=1)
            gl = jnp.dot(x_ref[...], rfull,
                         preferred_element_type=jnp.float32,
                         precision=lax.Precision.HIGHEST)
            gr = jnp.dot(xrem_ref[...], rfull,
                         preferred_element_type=jnp.float32,
                         precision=lax.Precision.HIGHEST)
            wdl_ref[...] = _top2_weights(gl)
            wdr_ref[...] = _top2_weights(gr)
            accl_ref[...] = jnp.zeros_like(accl_ref)
            accr_ref[...] = jnp.zeros_like(accr_ref)

        w1 = w1_ref[0]
        w2 = w2_ref[0]
        colmask = lax.broadcasted_iota(jnp.int32, (1, E), 1) == e
        wl = jnp.sum(jnp.where(colmask, wdl_ref[...], 0.0), axis=1,
                     keepdims=True)
        wr = jnp.sum(jnp.where(colmask, wdr_ref[...], 0.0), axis=1,
                     keepdims=True)

        hl = jnp.maximum(
            jnp.dot(x_ref[...], w1, preferred_element_type=jnp.float32), 0.0)
        accl_ref[...] += wl * jnp.dot(
            hl, w2, preferred_element_type=jnp.float32)
        hr = jnp.maximum(
            jnp.dot(xrem_ref[...], w1, preferred_element_type=jnp.float32),
            0.0)
        accr_ref[...] += wr * jnp.dot(
            hr, w2, preferred_element_type=jnp.float32)

        @pl.when(e == E_LOC - 1)
        def _combine():
            p_rdma = pltpu.make_async_remote_copy(
                src_ref=accr_ref, dst_ref=pp_ref,
                send_sem=send_sems.at[2], recv_sem=recv_sems.at[2],
                device_id=peer, device_id_type=pl.DeviceIdType.MESH,
            )
            p_rdma.start()
            p_rdma.wait()
            out_ref[...] = accl_ref[...] + pp_ref[...]

    return pl.pallas_call(
        body,
        grid=(E_LOC,),
        out_shape=jax.ShapeDtypeStruct((T_LOC, D), jnp.float32),
        in_specs=[
            pl.BlockSpec((T_LOC, D), lambda e: (0, 0)),
            pl.BlockSpec((D, E_LOC), lambda e: (0, 0)),
            pl.BlockSpec((1, D, F), lambda e: (e, 0, 0)),
            pl.BlockSpec((1, F, D), lambda e: (e, 0, 0)),
        ],
        out_specs=pl.BlockSpec((T_LOC, D), lambda e: (0, 0)),
        scratch_shapes=[
            pltpu.VMEM((T_LOC, D), jnp.float32),
            pltpu.VMEM((D, E_LOC), jnp.float32),
            pltpu.VMEM((T_LOC, E), jnp.float32),
            pltpu.VMEM((T_LOC, E), jnp.float32),
            pltpu.VMEM((T_LOC, D), jnp.float32),
            pltpu.VMEM((T_LOC, D), jnp.float32),
            pltpu.VMEM((T_LOC, D), jnp.float32),
            pltpu.SemaphoreType.DMA((3,)),
            pltpu.SemaphoreType.DMA((3,)),
        ],
        compiler_params=pltpu.CompilerParams(
            collective_id=0,
            dimension_semantics=("arbitrary",),
        ),
    )(x, router, W1, W2)


# baseline (device time: 114359 ns/iter reference)
import jax
import jax.numpy as jnp
from jax import lax
from jax.experimental import pallas as pl
from jax.experimental.pallas import tpu as pltpu

T_LOC = 512
D = 1024
F = 2048
E_LOC = 4
E = 8


def _top2_weights(gates):
    col = lax.broadcasted_iota(jnp.int32, gates.shape, 1)
    a1 = jnp.argmax(gates, axis=1)[:, None]
    m1 = col == a1
    v1 = jnp.max(gates, axis=1, keepdims=True)
    g2 = jnp.where(m1, -1e30, gates)
    a2 = jnp.argmax(g2, axis=1)[:, None]
    m2 = col == a2
    v2 = jnp.max(g2, axis=1, keepdims=True)
    z = jnp.exp(v2 - v1)
    w_top = 1.0 / (1.0 + z)
    w_sec = z / (1.0 + z)
    zero = jnp.zeros_like(gates)
    return jnp.where(m1, w_top, zero) + jnp.where(m2, w_sec, zero)


def kernel(x, router, W1, W2):
    def body(
        x_ref, r_ref, w1_ref, w2_ref, out_ref,
        xrem_ref, rrem_ref, wdl_ref, wdr_ref,
        accl_ref, accr_ref, pp_ref, send_sems, recv_sems,
    ):
        e = pl.program_id(0)
        my_x = lax.axis_index("x")
        my_y = lax.axis_index("y")
        peer = (my_x, 1 - my_y)

        @pl.when(e == 0)
        def _dispatch():
            barrier = pltpu.get_barrier_semaphore()
            pl.semaphore_signal(
                barrier, inc=1, device_id=peer,
                device_id_type=pl.DeviceIdType.MESH,
            )
            pl.semaphore_wait(barrier, 1)

            x_rdma = pltpu.make_async_remote_copy(
                src_ref=x_ref, dst_ref=xrem_ref,
                send_sem=send_sems.at[0], recv_sem=recv_sems.at[0],
                device_id=peer, device_id_type=pl.DeviceIdType.MESH,
            )
            r_rdma = pltpu.make_async_remote_copy(
                src_ref=r_ref, dst_ref=rrem_ref,
                send_sem=send_sems.at[1], recv_sem=recv_sems.at[1],
                device_id=peer, device_id_type=pl.DeviceIdType.MESH,
            )
            x_rdma.start()
            r_rdma.start()
            x_rdma.wait()
            r_rdma.wait()

            rfull = jnp.concatenate([r_ref[...], rrem_ref[...]], axis=1)
            gl = jnp.dot(x_ref[...], rfull,
                         preferred_element_type=jnp.float32,
                         precision=lax.Precision.HIGHEST)
            gr = jnp.dot(xrem_ref[...], rfull,
                         preferred_element_type=jnp.float32,
                         precision=lax.Precision.HIGHEST)
            wdl_ref[...] = _top2_weights(gl)
            wdr_ref[...] = _top2_weights(gr)
            accl_ref[...] = jnp.zeros_like(accl_ref)
            accr_ref[...] = jnp.zeros_like(accr_ref)

        w1 = w1_ref[0]
        w2 = w2_ref[0]
        colmask = lax.broadcasted_iota(jnp.int32, (1, E), 1) == e
        wl = jnp.sum(jnp.where(colmask, wdl_ref[...], 0.0), axis=1,
                     keepdims=True)
        wr = jnp.sum(jnp.where(colmask, wdr_ref[...], 0.0), axis=1,
                     keepdims=True)

        hl = jnp.maximum(
            jnp.dot(x_ref[...], w1, preferred_element_type=jnp.float32), 0.0)
        accl_ref[...] += wl * jnp.dot(
            hl, w2, preferred_element_type=jnp.float32)
        hr = jnp.maximum(
            jnp.dot(xrem_ref[...], w1, preferred_element_type=jnp.float32),
            0.0)
        accr_ref[...] += wr * jnp.dot(
            hr, w2, preferred_element_type=jnp.float32)

        @pl.when(e == E_LOC - 1)
        def _combine():
            p_rdma = pltpu.make_async_remote_copy(
                src_ref=accr_ref, dst_ref=pp_ref,
                send_sem=send_sems.at[2], recv_sem=recv_sems.at[2],
                device_id=peer, device_id_type=pl.DeviceIdType.MESH,
            )
            p_rdma.start()
            p_rdma.wait()
            out_ref[...] = accl_ref[...] + pp_ref[...]

    return pl.pallas_call(
        body,
        grid=(E_LOC,),
        out_shape=jax.ShapeDtypeStruct((T_LOC, D), jnp.float32),
        in_specs=[
            pl.BlockSpec((T_LOC, D), lambda e: (0, 0)),
            pl.BlockSpec((D, E_LOC), lambda e: (0, 0)),
            pl.BlockSpec((1, D, F), lambda e: (e, 0, 0)),
            pl.BlockSpec((1, F, D), lambda e: (e, 0, 0)),
        ],
        out_specs=pl.BlockSpec((T_LOC, D), lambda e: (0, 0)),
        scratch_shapes=[
            pltpu.VMEM((T_LOC, D), jnp.float32),
            pltpu.VMEM((D, E_LOC), jnp.float32),
            pltpu.VMEM((T_LOC, E), jnp.float32),
            pltpu.VMEM((T_LOC, E), jnp.float32),
            pltpu.VMEM((T_LOC, D), jnp.float32),
            pltpu.VMEM((T_LOC, D), jnp.float32),
            pltpu.VMEM((T_LOC, D), jnp.float32),
            pltpu.SemaphoreType.DMA((3,)),
            pltpu.SemaphoreType.DMA((3,)),
        ],
        compiler_params=pltpu.CompilerParams(
            collective_id=0,
            dimension_semantics=("arbitrary",),
            vmem_limit_bytes=110 * 1024 * 1024,
        ),
    )(x, router, W1, W2)


# device time: 106121 ns/iter; 1.0776x vs baseline; 1.0776x over previous
import jax
import jax.numpy as jnp
from jax import lax
from jax.experimental import pallas as pl
from jax.experimental.pallas import tpu as pltpu

T_LOC = 512
H = 256
D = 1024
F = 2048
E_LOC = 4
E = 8


def _top2_weights(gates):
    col = lax.broadcasted_iota(jnp.int32, gates.shape, 1)
    a1 = jnp.argmax(gates, axis=1)[:, None]
    m1 = col == a1
    v1 = jnp.max(gates, axis=1, keepdims=True)
    g2 = jnp.where(m1, -1e30, gates)
    a2 = jnp.argmax(g2, axis=1)[:, None]
    m2 = col == a2
    v2 = jnp.max(g2, axis=1, keepdims=True)
    z = jnp.exp(v2 - v1)
    w_top = 1.0 / (1.0 + z)
    w_sec = z / (1.0 + z)
    zero = jnp.zeros_like(gates)
    return jnp.where(m1, w_top, zero) + jnp.where(m2, w_sec, zero)


def kernel(x, router, W1, W2):
    def body(
        x_ref, r_ref, w1_ref, w2_ref, out_ref,
        xrem_ref, rrem_ref, wdl_ref, wdr_ref,
        accl_ref, accr_ref, pp_ref, qq_ref, s_ref,
        send_sems, recv_sems,
    ):
        i = pl.program_id(0)
        my_x = lax.axis_index("x")
        my_y = lax.axis_index("y")
        ypeer = (my_x, 1 - my_y)
        xpeer = (1 - my_x, my_y)
        c = my_x * H

        def x_dispatch():
            return pltpu.make_async_remote_copy(
                src_ref=x_ref.at[pl.ds(c, H), :], dst_ref=xrem_ref,
                send_sem=send_sems.at[0], recv_sem=recv_sems.at[0],
                device_id=ypeer, device_id_type=pl.DeviceIdType.MESH,
            )

        @pl.when(i == 0)
        def _start():
            barrier = pltpu.get_barrier_semaphore()
            for nbr in (ypeer, xpeer):
                pl.semaphore_signal(
                    barrier, inc=1, device_id=nbr,
                    device_id_type=pl.DeviceIdType.MESH,
                )
            pl.semaphore_wait(barrier, 2)

            x_dispatch().start()
            r_rdma = pltpu.make_async_remote_copy(
                src_ref=r_ref, dst_ref=rrem_ref,
                send_sem=send_sems.at[1], recv_sem=recv_sems.at[1],
                device_id=ypeer, device_id_type=pl.DeviceIdType.MESH,
            )
            r_rdma.start()
            r_rdma.wait()

            rfull = jnp.concatenate([r_ref[...], rrem_ref[...]], axis=1)
            gl = jnp.dot(x_ref[pl.ds(c, H), :], rfull,
                         preferred_element_type=jnp.float32,
                         precision=lax.Precision.HIGHEST)
            wdl_ref[...] = _top2_weights(gl)
            accl_ref[...] = jnp.zeros_like(accl_ref)

        @pl.when(i == E_LOC)
        def _dispatch_done():
            xd = x_dispatch()
            xd.wait_send()
            xd.wait_recv()
            rfull = jnp.concatenate([r_ref[...], rrem_ref[...]], axis=1)
            gr = jnp.dot(xrem_ref[...], rfull,
                         preferred_element_type=jnp.float32,
                         precision=lax.Precision.HIGHEST)
            wdr_ref[...] = _top2_weights(gr)
            accr_ref[...] = jnp.zeros_like(accr_ref)

        e = jnp.where(i < E_LOC, i, i - E_LOC)
        colmask = lax.broadcasted_iota(jnp.int32, (1, E), 1) == e
        w1 = w1_ref[0]
        w2 = w2_ref[0]

        @pl.when(i < E_LOC)
        def _local_rows():
            wl = jnp.sum(jnp.where(colmask, wdl_ref[...], 0.0), axis=1,
                         keepdims=True)
            hl = jnp.maximum(
                jnp.dot(x_ref[pl.ds(c, H), :], w1,
                        preferred_element_type=jnp.float32), 0.0)
            accl_ref[...] += wl * jnp.dot(
                hl, w2, preferred_element_type=jnp.float32)

        @pl.when(i >= E_LOC)
        def _peer_rows():
            wr = jnp.sum(jnp.where(colmask, wdr_ref[...], 0.0), axis=1,
                         keepdims=True)
            hr = jnp.maximum(
                jnp.dot(xrem_ref[...], w1,
                        preferred_element_type=jnp.float32), 0.0)
            accr_ref[...] += wr * jnp.dot(
                hr, w2, preferred_element_type=jnp.float32)

        @pl.when(i == 2 * E_LOC - 1)
        def _combine():
            py = pltpu.make_async_remote_copy(
                src_ref=accr_ref, dst_ref=pp_ref,
                send_sem=send_sems.at[2], recv_sem=recv_sems.at[2],
                device_id=ypeer, device_id_type=pl.DeviceIdType.MESH,
            )
            py.start()
            py.wait()
            s_ref[...] = accl_ref[...] + pp_ref[...]
            px = pltpu.make_async_remote_copy(
                src_ref=s_ref, dst_ref=qq_ref,
                send_sem=send_sems.at[3], recv_sem=recv_sems.at[3],
                device_id=xpeer, device_id_type=pl.DeviceIdType.MESH,
            )
            px.start()
            px.wait()
            out_ref[pl.ds(c, H), :] = s_ref[...]
            out_ref[pl.ds((1 - my_x) * H, H), :] = qq_ref[...]

    return pl.pallas_call(
        body,
        grid=(2 * E_LOC,),
        out_shape=jax.ShapeDtypeStruct((T_LOC, D), jnp.float32),
        in_specs=[
            pl.BlockSpec((T_LOC, D), lambda i: (0, 0)),
            pl.BlockSpec((D, E_LOC), lambda i: (0, 0)),
            pl.BlockSpec((1, D, F), lambda i: (i % E_LOC, 0, 0)),
            pl.BlockSpec((1, F, D), lambda i: (i % E_LOC, 0, 0)),
        ],
        out_specs=pl.BlockSpec((T_LOC, D), lambda i: (0, 0)),
        scratch_shapes=[
            pltpu.VMEM((H, D), jnp.float32),
            pltpu.VMEM((D, E_LOC), jnp.float32),
            pltpu.VMEM((H, E), jnp.float32),
            pltpu.VMEM((H, E), jnp.float32),
            pltpu.VMEM((H, D), jnp.float32),
            pltpu.VMEM((H, D), jnp.float32),
            pltpu.VMEM((H, D), jnp.float32),
            pltpu.VMEM((H, D), jnp.float32),
            pltpu.VMEM((H, D), jnp.float32),
            pltpu.SemaphoreType.DMA((4,)),
            pltpu.SemaphoreType.DMA((4,)),
        ],
        compiler_params=pltpu.CompilerParams(
            collective_id=0,
            dimension_semantics=("arbitrary",),
            vmem_limit_bytes=110 * 1024 * 1024,
        ),
    )(x, router, W1, W2)


# device time: 88094 ns/iter; 1.2981x vs baseline; 1.2046x over previous
import jax
import jax.numpy as jnp
from jax import lax
from jax.experimental import pallas as pl
from jax.experimental.pallas import tpu as pltpu

T_LOC = 512
H = 256
D = 1024
F = 2048
E_LOC = 4
E = 8


def _top2_weights(gates):
    col = lax.broadcasted_iota(jnp.int32, gates.shape, 1)
    a1 = jnp.argmax(gates, axis=1)[:, None]
    m1 = col == a1
    v1 = jnp.max(gates, axis=1, keepdims=True)
    g2 = jnp.where(m1, -1e30, gates)
    a2 = jnp.argmax(g2, axis=1)[:, None]
    m2 = col == a2
    v2 = jnp.max(g2, axis=1, keepdims=True)
    z = jnp.exp(v2 - v1)
    w_top = 1.0 / (1.0 + z)
    w_sec = z / (1.0 + z)
    zero = jnp.zeros_like(gates)
    return jnp.where(m1, w_top, zero) + jnp.where(m2, w_sec, zero)


def kernel(x, router, W1, W2):
    def body(
        x_ref, r_ref, w1_ref, w2_ref, out_ref,
        xrem_ref, rrem_ref, xlb_ref, xrb_ref, wdl_ref, wdr_ref,
        accl_ref, accr_ref, pp_ref, qq_ref, s_ref,
        send_sems, recv_sems,
    ):
        e = pl.program_id(0)
        my_x = lax.axis_index("x")
        my_y = lax.axis_index("y")
        ypeer = (my_x, 1 - my_y)
        xpeer = (1 - my_x, my_y)
        c = my_x * H

        @pl.when(e == 0)
        def _start():
            barrier = pltpu.get_barrier_semaphore()
            for nbr in (ypeer, xpeer):
                pl.semaphore_signal(
                    barrier, inc=1, device_id=nbr,
                    device_id_type=pl.DeviceIdType.MESH,
                )
            pl.semaphore_wait(barrier, 2)

            x_rdma = pltpu.make_async_remote_copy(
                src_ref=x_ref.at[pl.ds(c, H), :], dst_ref=xrem_ref,
                send_sem=send_sems.at[0], recv_sem=recv_sems.at[0],
                device_id=ypeer, device_id_type=pl.DeviceIdType.MESH,
            )
            x_rdma.start()
            r_rdma = pltpu.make_async_remote_copy(
                src_ref=r_ref, dst_ref=rrem_ref,
                send_sem=send_sems.at[1], recv_sem=recv_sems.at[1],
                device_id=ypeer, device_id_type=pl.DeviceIdType.MESH,
            )
            r_rdma.start()
            r_rdma.wait()

            rfull = jnp.concatenate([r_ref[...], rrem_ref[...]], axis=1)
            xl = x_ref[pl.ds(c, H), :]
            gl = jnp.dot(xl, rfull, preferred_element_type=jnp.float32,
                         precision=lax.Precision.HIGHEST)
            wdl_ref[...] = _top2_weights(gl)
            xlb_ref[...] = xl.astype(jnp.bfloat16)
            accl_ref[...] = jnp.zeros_like(accl_ref)

        colmask = lax.broadcasted_iota(jnp.int32, (1, E), 1) == e
        w1b = w1_ref[0].astype(jnp.bfloat16)
        w2b = w2_ref[0].astype(jnp.bfloat16)

        def contrib(xb):
            h = jnp.maximum(
                jnp.dot(xb, w1b, preferred_element_type=jnp.float32), 0.0)
            return jnp.dot(h.astype(jnp.bfloat16), w2b,
                           preferred_element_type=jnp.float32)

        wl = jnp.sum(jnp.where(colmask, wdl_ref[...], 0.0), axis=1,
                     keepdims=True)
        accl_ref[...] += wl * contrib(xlb_ref[...])

        @pl.when(e == 0)
        def _dispatch_done():
            xd = pltpu.make_async_remote_copy(
                src_ref=x_ref.at[pl.ds(c, H), :], dst_ref=xrem_ref,
                send_sem=send_sems.at[0], recv_sem=recv_sems.at[0],
                device_id=ypeer, device_id_type=pl.DeviceIdType.MESH,
            )
            xd.wait_send()
            xd.wait_recv()
            rfull = jnp.concatenate([r_ref[...], rrem_ref[...]], axis=1)
            gr = jnp.dot(xrem_ref[...], rfull,
                         preferred_element_type=jnp.float32,
                         precision=lax.Precision.HIGHEST)
            wdr_ref[...] = _top2_weights(gr)
            xrb_ref[...] = xrem_ref[...].astype(jnp.bfloat16)
            accr_ref[...] = jnp.zeros_like(accr_ref)

        wr = jnp.sum(jnp.where(colmask, wdr_ref[...], 0.0), axis=1,
                     keepdims=True)
        accr_ref[...] += wr * contrib(xrb_ref[...])

        @pl.when(e == E_LOC - 1)
        def _combine():
            py = pltpu.make_async_remote_copy(
                src_ref=accr_ref, dst_ref=pp_ref,
                send_sem=send_sems.at[2], recv_sem=recv_sems.at[2],
                device_id=ypeer, device_id_type=pl.DeviceIdType.MESH,
            )
            py.start()
            py.wait()
            s_ref[...] = accl_ref[...] + pp_ref[...]
            px = pltpu.make_async_remote_copy(
                src_ref=s_ref, dst_ref=qq_ref,
                send_sem=send_sems.at[3], recv_sem=recv_sems.at[3],
                device_id=xpeer, device_id_type=pl.DeviceIdType.MESH,
            )
            px.start()
            px.wait()
            out_ref[pl.ds(c, H), :] = s_ref[...]
            out_ref[pl.ds((1 - my_x) * H, H), :] = qq_ref[...]

    return pl.pallas_call(
        body,
        grid=(E_LOC,),
        out_shape=jax.ShapeDtypeStruct((T_LOC, D), jnp.float32),
        in_specs=[
            pl.BlockSpec((T_LOC, D), lambda i: (0, 0)),
            pl.BlockSpec((D, E_LOC), lambda i: (0, 0)),
            pl.BlockSpec((1, D, F), lambda i: (i, 0, 0)),
            pl.BlockSpec((1, F, D), lambda i: (i, 0, 0)),
        ],
        out_specs=pl.BlockSpec((T_LOC, D), lambda i: (0, 0)),
        scratch_shapes=[
            pltpu.VMEM((H, D), jnp.float32),
            pltpu.VMEM((D, E_LOC), jnp.float32),
            pltpu.VMEM((H, D), jnp.bfloat16),
            pltpu.VMEM((H, D), jnp.bfloat16),
            pltpu.VMEM((H, E), jnp.float32),
            pltpu.VMEM((H, E), jnp.float32),
            pltpu.VMEM((H, D), jnp.float32),
            pltpu.VMEM((H, D), jnp.float32),
            pltpu.VMEM((H, D), jnp.float32),
            pltpu.VMEM((H, D), jnp.float32),
            pltpu.VMEM((H, D), jnp.float32),
            pltpu.SemaphoreType.DMA((4,)),
            pltpu.SemaphoreType.DMA((4,)),
        ],
        compiler_params=pltpu.CompilerParams(
            collective_id=0,
            dimension_semantics=("arbitrary",),
            vmem_limit_bytes=110 * 1024 * 1024,
        ),
    )(x, router, W1, W2)


# device time: 70717 ns/iter; 1.6171x vs baseline; 1.2457x over previous
import jax
import jax.numpy as jnp
from jax import lax
from jax.experimental import pallas as pl
from jax.experimental.pallas import tpu as pltpu

T_LOC = 512
H = 256
CH = 128
D = 1024
F = 2048
E_LOC = 4
E = 8


def _top2_weights(gates):
    col = lax.broadcasted_iota(jnp.int32, gates.shape, 1)
    a1 = jnp.argmax(gates, axis=1)[:, None]
    m1 = col == a1
    v1 = jnp.max(gates, axis=1, keepdims=True)
    g2 = jnp.where(m1, -1e30, gates)
    a2 = jnp.argmax(g2, axis=1)[:, None]
    m2 = col == a2
    v2 = jnp.max(g2, axis=1, keepdims=True)
    z = jnp.exp(v2 - v1)
    w_top = 1.0 / (1.0 + z)
    w_sec = z / (1.0 + z)
    zero = jnp.zeros_like(gates)
    return jnp.where(m1, w_top, zero) + jnp.where(m2, w_sec, zero)


def kernel(x, router, W1, W2):
    def body(
        x_ref, r_ref, w1_ref, w2_ref, out_ref,
        xsb_ref, xremb_ref, rrem_ref, wdl_ref, wdr_ref,
        accl_ref, accr_ref, accrb_ref, pp_ref, qq_ref,
        send_sems, recv_sems,
    ):
        e = pl.program_id(0)
        my_x = lax.axis_index("x")
        my_y = lax.axis_index("y")
        ypeer = (my_x, 1 - my_y)
        xpeer = (1 - my_x, my_y)
        c = my_x * H

        def disp():
            return pltpu.make_async_remote_copy(
                src_ref=xsb_ref, dst_ref=xremb_ref,
                send_sem=send_sems.at[0], recv_sem=recv_sems.at[0],
                device_id=ypeer, device_id_type=pl.DeviceIdType.MESH,
            )

        def wd_send():
            return pltpu.make_async_remote_copy(
                src_ref=wdl_ref, dst_ref=wdr_ref,
                send_sem=send_sems.at[2], recv_sem=recv_sems.at[2],
                device_id=ypeer, device_id_type=pl.DeviceIdType.MESH,
            )

        def comb_y(j):
            return pltpu.make_async_remote_copy(
                src_ref=accrb_ref.at[pl.ds(j * CH, CH), :],
                dst_ref=pp_ref.at[pl.ds(j * CH, CH), :],
                send_sem=send_sems.at[3 + j], recv_sem=recv_sems.at[3 + j],
                device_id=ypeer, device_id_type=pl.DeviceIdType.MESH,
            )

        def comb_x(j):
            return pltpu.make_async_remote_copy(
                src_ref=xsb_ref.at[pl.ds(j * CH, CH), :],
                dst_ref=qq_ref.at[pl.ds(j * CH, CH), :],
                send_sem=send_sems.at[5 + j], recv_sem=recv_sems.at[5 + j],
                device_id=xpeer, device_id_type=pl.DeviceIdType.MESH,
            )

        w1b = w1_ref[0]
        w2b = w2_ref[0]

        def contrib(xrows):
            h = jnp.maximum(
                jnp.dot(xrows, w1b, preferred_element_type=jnp.float32), 0.0)
            return jnp.dot(h, w2b, preferred_element_type=jnp.float32)

        def wcol(wd_ref, col_idx):
            colmask = lax.broadcasted_iota(jnp.int32, (1, E), 1) == col_idx
            return jnp.sum(jnp.where(colmask, wd_ref[...], 0.0),
                           axis=1, keepdims=True)

        @pl.when(e == 0)
        def _start():
            barrier = pltpu.get_barrier_semaphore()
            for nbr in (ypeer, xpeer):
                pl.semaphore_signal(
                    barrier, inc=1, device_id=nbr,
                    device_id_type=pl.DeviceIdType.MESH,
                )
            pl.semaphore_wait(barrier, 2)

            xsb_ref[...] = x_ref[pl.ds(c, H), :].astype(jnp.bfloat16)
            disp().start()
            r_rdma = pltpu.make_async_remote_copy(
                src_ref=r_ref, dst_ref=rrem_ref,
                send_sem=send_sems.at[1], recv_sem=recv_sems.at[1],
                device_id=ypeer, device_id_type=pl.DeviceIdType.MESH,
            )
            r_rdma.start()
            r_rdma.wait()

            rfull = jnp.concatenate([r_ref[...], rrem_ref[...]], axis=1)
            gl = jnp.dot(x_ref[pl.ds(c, H), :], rfull,
                         preferred_element_type=jnp.float32,
                         precision=lax.Precision.HIGHEST)
            wdl_ref[...] = _top2_weights(gl)
            wd_send().start()
            accl_ref[...] = jnp.zeros_like(accl_ref)

        accl_ref[...] += wcol(wdl_ref, e) * contrib(x_ref[pl.ds(c, H), :])

        @pl.when(e == 0)
        def _dispatch_done():
            d = disp()
            d.wait_send()
            d.wait_recv()
            w = wd_send()
            w.wait_send()
            w.wait_recv()
            accr_ref[...] = jnp.zeros_like(accr_ref)

        accr_ref[...] += wcol(wdr_ref, e + E_LOC) * contrib(xremb_ref[...])

        @pl.when(e == E_LOC - 1)
        def _combine():
            for j in range(2):
                accrb_ref[pl.ds(j * CH, CH), :] = (
                    accr_ref[pl.ds(j * CH, CH), :].astype(jnp.bfloat16))
                comb_y(j).start()
            for j in range(2):
                cy = comb_y(j)
                cy.wait_send()
                cy.wait_recv()
                m = (accl_ref[pl.ds(j * CH, CH), :]
                     + pp_ref[pl.ds(j * CH, CH), :].astype(jnp.float32))
                out_ref[pl.ds(c + j * CH, CH), :] = m
                xsb_ref[pl.ds(j * CH, CH), :] = m.astype(jnp.bfloat16)
                comb_x(j).start()
            for j in range(2):
                cx = comb_x(j)
                cx.wait_send()
                cx.wait_recv()
            out_ref[pl.ds((1 - my_x) * H, H), :] = (
                qq_ref[...].astype(jnp.float32))

    return pl.pallas_call(
        body,
        grid=(E_LOC,),
        out_shape=jax.ShapeDtypeStruct((T_LOC, D), jnp.float32),
        in_specs=[
            pl.BlockSpec((T_LOC, D), lambda i: (0, 0)),
            pl.BlockSpec((D, E_LOC), lambda i: (0, 0)),
            pl.BlockSpec((1, D, F), lambda i: (i, 0, 0)),
            pl.BlockSpec((1, F, D), lambda i: (i, 0, 0)),
        ],
        out_specs=pl.BlockSpec((T_LOC, D), lambda i: (0, 0)),
        scratch_shapes=[
            pltpu.VMEM((H, D), jnp.bfloat16),
            pltpu.VMEM((H, D), jnp.bfloat16),
            pltpu.VMEM((D, E_LOC), jnp.float32),
            pltpu.VMEM((H, E), jnp.float32),
            pltpu.VMEM((H, E), jnp.float32),
            pltpu.VMEM((H, D), jnp.float32),
            pltpu.VMEM((H, D), jnp.float32),
            pltpu.VMEM((H, D), jnp.bfloat16),
            pltpu.VMEM((H, D), jnp.bfloat16),
            pltpu.VMEM((H, D), jnp.bfloat16),
            pltpu.SemaphoreType.DMA((7,)),
            pltpu.SemaphoreType.DMA((7,)),
        ],
        compiler_params=pltpu.CompilerParams(
            collective_id=0,
            dimension_semantics=("arbitrary",),
            vmem_limit_bytes=63 * 1024 * 1024,
        ),
    )(x, router, W1, W2)
